# baseline (speedup 1.0000x reference)
import functools

import jax
import jax.numpy as jnp
from jax import lax
from jax.experimental import pallas as pl
from jax.experimental.pallas import tpu as pltpu
from jax.experimental.pallas import tpu_sc as plsc

_CHUNK = 128
_NBUF = 3
_IBUF = 4


def _sc_scatter(x, srcdst, zeros, e):
    n, d = x.shape
    info = plsc.get_sparse_core_info()
    nc, ns = info.num_cores, info.num_subcores
    nw = nc * ns
    n_chunks = e // _CHUNK
    rows_per_tile = (n // ns) // 8 * 8
    tail_base = ns * rows_per_tile
    tail_rows = n - tail_base

    mesh = plsc.VectorSubcoreMesh(core_axis_name="c", subcore_axis_name="s")

    @functools.partial(
        pl.kernel,
        out_type=jax.ShapeDtypeStruct((nc, n, d), jnp.float32),
        mesh=mesh,
        scratch_types=[
            pltpu.VMEM((_IBUF, _CHUNK), jnp.int32),
            pltpu.VMEM((_IBUF, _CHUNK), jnp.int32),
            pltpu.VMEM((_NBUF, _CHUNK, d), jnp.float32),
            pltpu.VMEM_SHARED((n, d), jnp.float32),
            pltpu.SemaphoreType.DMA,
            pltpu.SemaphoreType.DMA,
            pltpu.SemaphoreType.DMA,
        ],
    )
    def sc_kernel(x_hbm, srcdst_hbm, zero_hbm, out_hbm,
                  src_v, dst_v, rows_v, acc_sh, isem, gsem, ssem):
        cid = lax.axis_index("c")
        sid = lax.axis_index("s")
        wid = sid * nc + cid

        n_mine = (n_chunks - wid + nw - 1) // nw

        def ebase(j):
            return (wid + j * nw) * _CHUNK

        def issue_idx(j):
            slot = j % _IBUF
            pltpu.async_copy(srcdst_hbm.at[pl.ds(ebase(j), _CHUNK)],
                             src_v.at[slot], isem)
            pltpu.async_copy(srcdst_hbm.at[pl.ds(e + ebase(j), _CHUNK)],
                             dst_v.at[slot], isem)

        def wait_idx(j):
            slot = j % _IBUF
            pltpu.make_async_copy(srcdst_hbm.at[pl.ds(ebase(j), _CHUNK)],
                                  src_v.at[slot], isem).wait()
            pltpu.make_async_copy(srcdst_hbm.at[pl.ds(e + ebase(j), _CHUNK)],
                                  dst_v.at[slot], isem).wait()

        def issue_gather(j):
            pltpu.async_copy(x_hbm.at[src_v.at[j % _IBUF]],
                             rows_v.at[j % _NBUF], gsem)

        def wait_gather(j):
            pltpu.make_async_copy(x_hbm.at[src_v.at[j % _IBUF]],
                                  rows_v.at[j % _NBUF], gsem).wait()

        def issue_scatter(j):
            pltpu.async_copy(rows_v.at[j % _NBUF],
                             acc_sh.at[dst_v.at[j % _IBUF]], ssem, add=True)

        def wait_scatter(j):
            pltpu.make_async_copy(rows_v.at[j % _NBUF],
                                  acc_sh.at[dst_v.at[j % _IBUF]], ssem).wait()

        @pl.when(n_mine > 0)
        def _():
            issue_idx(0)
            wait_idx(0)
            issue_gather(0)

            @pl.when(n_mine > 1)
            def _():
                issue_idx(1)
                wait_idx(1)
                issue_gather(1)

            @pl.when(n_mine > 2)
            def _():
                issue_idx(2)

        pltpu.sync_copy(zero_hbm.at[pl.ds(0, rows_per_tile)],
                        acc_sh.at[pl.ds(sid * rows_per_tile, rows_per_tile)])

        @pl.when(sid == ns - 1)
        def _():
            pltpu.sync_copy(zero_hbm.at[pl.ds(0, tail_rows)],
                            acc_sh.at[pl.ds(tail_base, tail_rows)])

        plsc.subcore_barrier()

        @pl.when(n_mine > 0)
        def _():
            def body(j, carry):
                @pl.when(j >= 1)
                def _():
                    wait_scatter(j - 1)

                @pl.when(j + 2 < n_mine)
                def _():
                    wait_idx(j + 2)
                    issue_gather(j + 2)

                wait_gather(j)
                issue_scatter(j)

                @pl.when(j + 3 < n_mine)
                def _():
                    issue_idx(j + 3)

                return carry

            lax.fori_loop(0, n_mine, body, 0)
            wait_scatter(n_mine - 1)

        plsc.subcore_barrier()
        pltpu.sync_copy(
            acc_sh.at[pl.ds(sid * rows_per_tile, rows_per_tile)],
            out_hbm.at[cid].at[pl.ds(sid * rows_per_tile, rows_per_tile)],
        )

        @pl.when(sid == ns - 1)
        def _():
            pltpu.sync_copy(
                acc_sh.at[pl.ds(tail_base, tail_rows)],
                out_hbm.at[cid].at[pl.ds(tail_base, tail_rows)],
            )

    return sc_kernel(x, srcdst, zeros)


def _tc_h0_body(x_ref, w0_ref, o_ref):
    o_ref[...] = jnp.dot(x_ref[...], w0_ref[...],
                         preferred_element_type=jnp.float32)


def _tc_h0(x, w0):
    n, d = x.shape
    d_out = w0.shape[1]
    bm = 1000
    return pl.pallas_call(
        _tc_h0_body,
        grid=(n // bm,),
        in_specs=[
            pl.BlockSpec((bm, d), lambda i: (i, 0)),
            pl.BlockSpec((d, d_out), lambda i: (0, 0)),
        ],
        out_specs=pl.BlockSpec((bm, d_out), lambda i: (i, 0)),
        out_shape=jax.ShapeDtypeStruct((n, d_out), jnp.float32),
    )(x, w0)


def _tc_body(h0_ref, p_ref, w1_ref, o_ref):
    agg = p_ref[0] + p_ref[1]
    h1 = jnp.dot(agg, w1_ref[...], preferred_element_type=jnp.float32)
    o_ref[...] = jnp.maximum(h0_ref[...] + h1, 0.0)


def _tc_combine(h0, partials, w1):
    n, d_out = h0.shape
    d = partials.shape[2]
    bm = 1000
    return pl.pallas_call(
        _tc_body,
        grid=(n // bm,),
        in_specs=[
            pl.BlockSpec((bm, d_out), lambda i: (i, 0)),
            pl.BlockSpec((2, bm, d), lambda i: (0, i, 0)),
            pl.BlockSpec((d, d_out), lambda i: (0, 0)),
        ],
        out_specs=pl.BlockSpec((bm, d_out), lambda i: (i, 0)),
        out_shape=jax.ShapeDtypeStruct((n, d_out), jnp.float32),
    )(h0, partials, w1)


def kernel(vertex_features, vertex_adjacency, w0, w1):
    x = vertex_features
    n, d = x.shape
    e = vertex_adjacency.shape[1]
    srcdst = vertex_adjacency.reshape(2 * e)
    zeros = jnp.zeros((n // 16 // 8 * 8, d), jnp.float32)
    partials = _sc_scatter(x, srcdst, zeros, e)
    h0 = _tc_h0(x, w0)
    return _tc_combine(h0, partials, w1)


# speedup vs baseline: 1.0253x; 1.0253x over previous
import functools

import jax
import jax.numpy as jnp
from jax import lax
from jax.experimental import pallas as pl
from jax.experimental.pallas import tpu as pltpu
from jax.experimental.pallas import tpu_sc as plsc

_CHUNK = 128
_NBUF = 3
_IBUF = 4


def _sc_scatter(x, srcdst, zeros, e):
    n, d = x.shape
    info = plsc.get_sparse_core_info()
    nc, ns = info.num_cores, info.num_subcores
    nw = nc * ns
    n_chunks = e // _CHUNK
    rows_per_tile = (n // ns) // 8 * 8
    tail_base = ns * rows_per_tile
    tail_rows = n - tail_base

    mesh = plsc.VectorSubcoreMesh(core_axis_name="c", subcore_axis_name="s")

    @functools.partial(
        pl.kernel,
        out_type=jax.ShapeDtypeStruct((nc, n, d), jnp.float32),
        mesh=mesh,
        scratch_types=[
            pltpu.VMEM((_IBUF, _CHUNK), jnp.int32),
            pltpu.VMEM((_IBUF, _CHUNK), jnp.int32),
            pltpu.VMEM((_NBUF, _CHUNK, d), jnp.float32),
            pltpu.VMEM_SHARED((n, d), jnp.float32),
            pltpu.SemaphoreType.DMA,
            pltpu.SemaphoreType.DMA,
            pltpu.SemaphoreType.DMA,
        ],
    )
    def sc_kernel(x_hbm, srcdst_hbm, zero_hbm, out_hbm,
                  src_v, dst_v, rows_v, acc_sh, isem, gsem, ssem):
        cid = lax.axis_index("c")
        sid = lax.axis_index("s")
        wid = sid * nc + cid

        n_mine = (n_chunks - wid + nw - 1) // nw

        def ebase(j):
            return (wid + j * nw) * _CHUNK

        def issue_idx(j):
            slot = j % _IBUF
            pltpu.async_copy(srcdst_hbm.at[pl.ds(ebase(j), _CHUNK)],
                             src_v.at[slot], isem)
            pltpu.async_copy(srcdst_hbm.at[pl.ds(e + ebase(j), _CHUNK)],
                             dst_v.at[slot], isem)

        def wait_idx(j):
            slot = j % _IBUF
            pltpu.make_async_copy(srcdst_hbm.at[pl.ds(ebase(j), _CHUNK)],
                                  src_v.at[slot], isem).wait()
            pltpu.make_async_copy(srcdst_hbm.at[pl.ds(e + ebase(j), _CHUNK)],
                                  dst_v.at[slot], isem).wait()

        def issue_gather(j):
            pltpu.async_copy(x_hbm.at[src_v.at[j % _IBUF]],
                             rows_v.at[j % _NBUF], gsem)

        def wait_gather(j):
            pltpu.make_async_copy(x_hbm.at[src_v.at[j % _IBUF]],
                                  rows_v.at[j % _NBUF], gsem).wait()

        def issue_scatter(j):
            pltpu.async_copy(rows_v.at[j % _NBUF],
                             acc_sh.at[dst_v.at[j % _IBUF]], ssem, add=True)

        def wait_scatter(j):
            pltpu.make_async_copy(rows_v.at[j % _NBUF],
                                  acc_sh.at[dst_v.at[j % _IBUF]], ssem).wait()

        @pl.when(n_mine > 0)
        def _():
            issue_idx(0)
            wait_idx(0)
            issue_gather(0)

            @pl.when(n_mine > 1)
            def _():
                issue_idx(1)
                wait_idx(1)
                issue_gather(1)

            @pl.when(n_mine > 2)
            def _():
                issue_idx(2)

        pltpu.sync_copy(zero_hbm.at[pl.ds(0, rows_per_tile)],
                        acc_sh.at[pl.ds(sid * rows_per_tile, rows_per_tile)])

        @pl.when(sid == ns - 1)
        def _():
            pltpu.sync_copy(zero_hbm.at[pl.ds(0, tail_rows)],
                            acc_sh.at[pl.ds(tail_base, tail_rows)])

        plsc.subcore_barrier()

        @pl.when(n_mine > 0)
        def _():
            def body(j, carry):
                @pl.when(j >= 1)
                def _():
                    wait_scatter(j - 1)

                @pl.when(j + 2 < n_mine)
                def _():
                    wait_idx(j + 2)
                    issue_gather(j + 2)

                wait_gather(j)
                issue_scatter(j)

                @pl.when(j + 3 < n_mine)
                def _():
                    issue_idx(j + 3)

                return carry

            lax.fori_loop(0, n_mine, body, 0)
            wait_scatter(n_mine - 1)

        plsc.subcore_barrier()
        pltpu.sync_copy(
            acc_sh.at[pl.ds(sid * rows_per_tile, rows_per_tile)],
            out_hbm.at[cid].at[pl.ds(sid * rows_per_tile, rows_per_tile)],
        )

        @pl.when(sid == ns - 1)
        def _():
            pltpu.sync_copy(
                acc_sh.at[pl.ds(tail_base, tail_rows)],
                out_hbm.at[cid].at[pl.ds(tail_base, tail_rows)],
            )

    return sc_kernel(x, srcdst, zeros)


def _tc_body(x_ref, p_ref, w0_ref, w1_ref, o_ref):
    h0 = jnp.dot(x_ref[...], w0_ref[...], preferred_element_type=jnp.float32)
    agg = p_ref[0] + p_ref[1]
    h1 = jnp.dot(agg, w1_ref[...], preferred_element_type=jnp.float32)
    o_ref[...] = jnp.maximum(h0 + h1, 0.0)


def _tc_combine(x, partials, w0, w1):
    n, d = x.shape
    d_out = w0.shape[1]
    bm = 1000
    return pl.pallas_call(
        _tc_body,
        grid=(n // bm,),
        in_specs=[
            pl.BlockSpec((bm, d), lambda i: (i, 0)),
            pl.BlockSpec((2, bm, d), lambda i: (0, i, 0)),
            pl.BlockSpec((d, d_out), lambda i: (0, 0)),
            pl.BlockSpec((d, d_out), lambda i: (0, 0)),
        ],
        out_specs=pl.BlockSpec((bm, d_out), lambda i: (i, 0)),
        out_shape=jax.ShapeDtypeStruct((n, d_out), jnp.float32),
    )(x, partials, w0, w1)


def kernel(vertex_features, vertex_adjacency, w0, w1):
    x = vertex_features
    n, d = x.shape
    e = vertex_adjacency.shape[1]
    srcdst = vertex_adjacency.reshape(2 * e)
    zeros = jnp.zeros((n // 16 // 8 * 8, d), jnp.float32)
    partials = _sc_scatter(x, srcdst, zeros, e)
    return _tc_combine(x, partials, w0, w1)


# speedup vs baseline: 1.0588x; 1.0327x over previous
import functools

import jax
import jax.numpy as jnp
from jax import lax
from jax.experimental import pallas as pl
from jax.experimental.pallas import tpu as pltpu
from jax.experimental.pallas import tpu_sc as plsc

_CHUNK = 128
_NBUF = 3
_IBUF = 4


def _sc_scatter(x, srcdst, zeros, e):
    n, d = x.shape
    info = plsc.get_sparse_core_info()
    nc, ns = info.num_cores, info.num_subcores
    nw = nc * ns
    n_chunks = e // _CHUNK
    rows_per_tile = (n // ns) // 8 * 8
    tail_base = ns * rows_per_tile
    tail_rows = n - tail_base

    mesh = plsc.VectorSubcoreMesh(core_axis_name="c", subcore_axis_name="s")

    @functools.partial(
        pl.kernel,
        out_type=jax.ShapeDtypeStruct((nc, n, d), jnp.float32),
        mesh=mesh,
        scratch_types=[
            pltpu.VMEM((_IBUF, _CHUNK), jnp.int32),
            pltpu.VMEM((_IBUF, _CHUNK), jnp.int32),
            pltpu.VMEM((_NBUF, _CHUNK, d), jnp.float32),
            pltpu.VMEM_SHARED((n, d), jnp.float32),
            pltpu.SemaphoreType.DMA,
            pltpu.SemaphoreType.DMA,
            pltpu.SemaphoreType.DMA,
        ],
    )
    def sc_kernel(x_hbm, srcdst_hbm, zero_hbm, out_hbm,
                  src_v, dst_v, rows_v, acc_sh, isem, gsem, ssem):
        cid = lax.axis_index("c")
        sid = lax.axis_index("s")
        wid = sid * nc + cid

        n_mine = (n_chunks - wid + nw - 1) // nw

        def ebase(j):
            return (wid + j * nw) * _CHUNK

        def issue_idx(j):
            slot = j % _IBUF
            pltpu.async_copy(srcdst_hbm.at[pl.ds(ebase(j), _CHUNK)],
                             src_v.at[slot], isem)
            pltpu.async_copy(srcdst_hbm.at[pl.ds(e + ebase(j), _CHUNK)],
                             dst_v.at[slot], isem)

        def wait_idx(j):
            slot = j % _IBUF
            pltpu.make_async_copy(srcdst_hbm.at[pl.ds(ebase(j), _CHUNK)],
                                  src_v.at[slot], isem).wait()
            pltpu.make_async_copy(srcdst_hbm.at[pl.ds(e + ebase(j), _CHUNK)],
                                  dst_v.at[slot], isem).wait()

        def issue_gather(j):
            pltpu.async_copy(x_hbm.at[src_v.at[j % _IBUF]],
                             rows_v.at[j % _NBUF], gsem)

        def wait_gather(j):
            pltpu.make_async_copy(x_hbm.at[src_v.at[j % _IBUF]],
                                  rows_v.at[j % _NBUF], gsem).wait()

        def issue_scatter(j):
            pltpu.async_copy(rows_v.at[j % _NBUF],
                             acc_sh.at[dst_v.at[j % _IBUF]], ssem, add=True)

        def wait_scatter(j):
            pltpu.make_async_copy(rows_v.at[j % _NBUF],
                                  acc_sh.at[dst_v.at[j % _IBUF]], ssem).wait()

        @pl.when(n_mine > 0)
        def _():
            issue_idx(0)
            wait_idx(0)
            issue_gather(0)

            @pl.when(n_mine > 1)
            def _():
                issue_idx(1)
                wait_idx(1)
                issue_gather(1)

            @pl.when(n_mine > 2)
            def _():
                issue_idx(2)

        pltpu.sync_copy(zero_hbm.at[pl.ds(0, rows_per_tile)],
                        acc_sh.at[pl.ds(sid * rows_per_tile, rows_per_tile)])

        @pl.when(sid == ns - 1)
        def _():
            pltpu.sync_copy(zero_hbm.at[pl.ds(0, tail_rows)],
                            acc_sh.at[pl.ds(tail_base, tail_rows)])

        plsc.subcore_barrier()

        @pl.when(n_mine > 0)
        def _():
            def body(j, carry):
                @pl.when(j >= 1)
                def _():
                    wait_scatter(j - 1)

                @pl.when(j + 2 < n_mine)
                def _():
                    wait_idx(j + 2)
                    issue_gather(j + 2)

                wait_gather(j)
                issue_scatter(j)

                @pl.when(j + 3 < n_mine)
                def _():
                    issue_idx(j + 3)

                return carry

            lax.fori_loop(0, n_mine, body, 0)
            wait_scatter(n_mine - 1)

        plsc.subcore_barrier()
        pltpu.sync_copy(
            acc_sh.at[pl.ds(sid * rows_per_tile, rows_per_tile)],
            out_hbm.at[cid].at[pl.ds(sid * rows_per_tile, rows_per_tile)],
        )

        @pl.when(sid == ns - 1)
        def _():
            pltpu.sync_copy(
                acc_sh.at[pl.ds(tail_base, tail_rows)],
                out_hbm.at[cid].at[pl.ds(tail_base, tail_rows)],
            )

    return sc_kernel(x, srcdst, zeros)


def _tc_body(x_ref, p_ref, w0_ref, w1_ref, o_ref):
    h0 = jnp.dot(x_ref[...], w0_ref[...], preferred_element_type=jnp.float32)
    agg = p_ref[0] + p_ref[1]
    h1 = jnp.dot(agg, w1_ref[...], preferred_element_type=jnp.float32)
    o_ref[...] = jnp.maximum(h0 + h1, 0.0)


def _tc_combine(x, partials, w0, w1):
    n, d = x.shape
    d_out = w0.shape[1]
    bm = 2000
    return pl.pallas_call(
        _tc_body,
        grid=(n // bm,),
        in_specs=[
            pl.BlockSpec((bm, d), lambda i: (i, 0)),
            pl.BlockSpec((2, bm, d), lambda i: (0, i, 0)),
            pl.BlockSpec((d, d_out), lambda i: (0, 0)),
            pl.BlockSpec((d, d_out), lambda i: (0, 0)),
        ],
        out_specs=pl.BlockSpec((bm, d_out), lambda i: (i, 0)),
        out_shape=jax.ShapeDtypeStruct((n, d_out), jnp.float32),
    )(x, partials, w0, w1)


def kernel(vertex_features, vertex_adjacency, w0, w1):
    x = vertex_features
    n, d = x.shape
    e = vertex_adjacency.shape[1]
    srcdst = vertex_adjacency.reshape(2 * e)
    zeros = jnp.zeros((n // 16 // 8 * 8, d), jnp.float32)
    partials = _sc_scatter(x, srcdst, zeros, e)
    return _tc_combine(x, partials, w0, w1)


# speedup vs baseline: 1.0605x; 1.0016x over previous
import functools

import jax
import jax.numpy as jnp
from jax import lax
from jax.experimental import pallas as pl
from jax.experimental.pallas import tpu as pltpu
from jax.experimental.pallas import tpu_sc as plsc

_CHUNK = 128
_NBUF = 3
_IBUF = 4


def _sc_scatter(x, srcdst, zeros, e):
    n, d = x.shape
    info = plsc.get_sparse_core_info()
    nc, ns = info.num_cores, info.num_subcores
    nw = nc * ns
    n_chunks = e // _CHUNK
    rows_per_tile = (n // ns) // 8 * 8
    tail_base = ns * rows_per_tile
    tail_rows = n - tail_base

    mesh = plsc.VectorSubcoreMesh(core_axis_name="c", subcore_axis_name="s")

    @functools.partial(
        pl.kernel,
        out_type=jax.ShapeDtypeStruct((nc, n, d), jnp.float32),
        mesh=mesh,
        scratch_types=[
            pltpu.VMEM((_IBUF, _CHUNK), jnp.int32),
            pltpu.VMEM((_IBUF, _CHUNK), jnp.int32),
            pltpu.VMEM((_NBUF, _CHUNK, d), jnp.float32),
            pltpu.VMEM_SHARED((n, d), jnp.float32),
            pltpu.SemaphoreType.DMA,
            pltpu.SemaphoreType.DMA,
            pltpu.SemaphoreType.DMA,
        ],
    )
    def sc_kernel(x_hbm, srcdst_hbm, zero_hbm, out_hbm,
                  src_v, dst_v, rows_v, acc_sh, isem, gsem, ssem):
        cid = lax.axis_index("c")
        sid = lax.axis_index("s")
        wid = sid * nc + cid

        n_mine = (n_chunks - wid + nw - 1) // nw

        def ebase(j):
            return (wid + j * nw) * _CHUNK

        def issue_idx(j):
            slot = j % _IBUF
            pltpu.async_copy(srcdst_hbm.at[pl.ds(ebase(j), _CHUNK)],
                             src_v.at[slot], isem)
            pltpu.async_copy(srcdst_hbm.at[pl.ds(e + ebase(j), _CHUNK)],
                             dst_v.at[slot], isem)

        def wait_idx(j):
            slot = j % _IBUF
            pltpu.make_async_copy(srcdst_hbm.at[pl.ds(ebase(j), _CHUNK)],
                                  src_v.at[slot], isem).wait()
            pltpu.make_async_copy(srcdst_hbm.at[pl.ds(e + ebase(j), _CHUNK)],
                                  dst_v.at[slot], isem).wait()

        def issue_gather(j):
            pltpu.async_copy(x_hbm.at[src_v.at[j % _IBUF]],
                             rows_v.at[j % _NBUF], gsem)

        def wait_gather(j):
            pltpu.make_async_copy(x_hbm.at[src_v.at[j % _IBUF]],
                                  rows_v.at[j % _NBUF], gsem).wait()

        def issue_scatter(j):
            pltpu.async_copy(rows_v.at[j % _NBUF],
                             acc_sh.at[dst_v.at[j % _IBUF]], ssem, add=True)

        def wait_scatter(j):
            pltpu.make_async_copy(rows_v.at[j % _NBUF],
                                  acc_sh.at[dst_v.at[j % _IBUF]], ssem).wait()

        @pl.when(n_mine > 0)
        def _():
            issue_idx(0)
            wait_idx(0)
            issue_gather(0)

            @pl.when(n_mine > 1)
            def _():
                issue_idx(1)
                wait_idx(1)
                issue_gather(1)

            @pl.when(n_mine > 2)
            def _():
                issue_idx(2)

        pltpu.sync_copy(zero_hbm.at[pl.ds(0, rows_per_tile)],
                        acc_sh.at[pl.ds(sid * rows_per_tile, rows_per_tile)])

        @pl.when(sid == ns - 1)
        def _():
            pltpu.sync_copy(zero_hbm.at[pl.ds(0, tail_rows)],
                            acc_sh.at[pl.ds(tail_base, tail_rows)])

        plsc.subcore_barrier()

        @pl.when(n_mine > 0)
        def _():
            def body(j, carry):
                @pl.when(j >= 1)
                def _():
                    wait_scatter(j - 1)

                @pl.when(j + 2 < n_mine)
                def _():
                    wait_idx(j + 2)
                    issue_gather(j + 2)

                wait_gather(j)
                issue_scatter(j)

                @pl.when(j + 3 < n_mine)
                def _():
                    issue_idx(j + 3)

                return carry

            lax.fori_loop(0, n_mine, body, 0)
            wait_scatter(n_mine - 1)

        plsc.subcore_barrier()
        pltpu.sync_copy(
            acc_sh.at[pl.ds(sid * rows_per_tile, rows_per_tile)],
            out_hbm.at[cid].at[pl.ds(sid * rows_per_tile, rows_per_tile)],
        )

        @pl.when(sid == ns - 1)
        def _():
            pltpu.sync_copy(
                acc_sh.at[pl.ds(tail_base, tail_rows)],
                out_hbm.at[cid].at[pl.ds(tail_base, tail_rows)],
            )

    return sc_kernel(x, srcdst, zeros)


def _tc_body(x_ref, p_ref, w0_ref, w1_ref, o_ref):
    h0 = jnp.dot(x_ref[...], w0_ref[...], preferred_element_type=jnp.float32)
    agg = p_ref[0] + p_ref[1]
    h1 = jnp.dot(agg, w1_ref[...], preferred_element_type=jnp.float32)
    o_ref[...] = jnp.maximum(h0 + h1, 0.0)


def _tc_combine(x, partials, w0, w1):
    n, d = x.shape
    d_out = w0.shape[1]
    bm = 5000
    return pl.pallas_call(
        _tc_body,
        grid=(n // bm,),
        in_specs=[
            pl.BlockSpec((bm, d), lambda i: (i, 0)),
            pl.BlockSpec((2, bm, d), lambda i: (0, i, 0)),
            pl.BlockSpec((d, d_out), lambda i: (0, 0)),
            pl.BlockSpec((d, d_out), lambda i: (0, 0)),
        ],
        out_specs=pl.BlockSpec((bm, d_out), lambda i: (i, 0)),
        out_shape=jax.ShapeDtypeStruct((n, d_out), jnp.float32),
    )(x, partials, w0, w1)


def kernel(vertex_features, vertex_adjacency, w0, w1):
    x = vertex_features
    n, d = x.shape
    e = vertex_adjacency.shape[1]
    srcdst = vertex_adjacency.reshape(2 * e)
    zeros = jnp.zeros((n // 16 // 8 * 8, d), jnp.float32)
    partials = _sc_scatter(x, srcdst, zeros, e)
    return _tc_combine(x, partials, w0, w1)
